# Initial kernel scaffold; baseline (speedup 1.0000x reference)
import jax
import jax.numpy as jnp
from jax.experimental import pallas as pl


def kernel(x, W_in, b_in, W_out, b_out):
    raise NotImplementedError("write your pallas kernel here")


# speedup vs baseline: 1.6885x; 1.6885x over previous
import functools

import jax
import jax.numpy as jnp
from jax.experimental import pallas as pl
from jax.experimental.pallas import tpu as pltpu

_DIM = 768
_NUM_Q = 4
_CB_DIM = 12
_HI_BITS = 5
_LO_BITS = 7
_EPS = 1e-5
_INV_TEMP = 100.0
_ENTROPY_W = 0.1
_COMMIT_W = 0.25


def _softplus(t):
    return jnp.maximum(t, 0.0) + jnp.log1p(jnp.exp(-jnp.abs(t)))


def _lfq_block_kernel(x_ref, win_ref, bin_ref, wout_ref, bout_ref,
                      out_ref, idx_ref, loss_ref,
                      avg_sc, sums_sc, *, n_rows_total, n_steps):
    i = pl.program_id(0)

    @pl.when(i == 0)
    def _init():
        avg_sc[...] = jnp.zeros_like(avg_sc)
        for _k in range(2):
            for _q in range(_NUM_Q):
                sums_sc[_k, _q] = jnp.float32(0.0)

    xb = x_ref[...]
    xp = jnp.dot(xb, win_ref[...],
                 preferred_element_type=jnp.float32) + bin_ref[...]

    jm = jax.lax.broadcasted_iota(jnp.int32, (1, _CB_DIM), 1)
    mask_row = jnp.right_shift(jnp.int32(2048), jm)
    hh = jax.lax.broadcasted_iota(jnp.int32, (1, 32), 1)
    hi_bit = [jnp.bitwise_and(
        jnp.right_shift(hh, (_HI_BITS - 1) - d), 1) == 1
        for d in range(_HI_BITS)]
    hl = jax.lax.broadcasted_iota(jnp.int32, (1, 128), 1)
    lo_bit = [jnp.bitwise_and(
        jnp.right_shift(hl, (_LO_BITS - 1) - d), 1) == 1
        for d in range(_LO_BITS)]

    r = xp
    qtot = jnp.zeros_like(xp)
    idx_cols = []
    for q in range(_NUM_Q):
        scale = 2.0 ** (-q)
        pos = r > 0
        qh = jnp.where(pos, scale, -scale)
        qst = r + (qh - r)

        idx_q = jnp.sum(jnp.where(pos, mask_row, 0), axis=1, keepdims=True)
        idx_cols.append(idx_q)

        t = (4.0 * _INV_TEMP * scale) * r
        lqn = -_softplus(t)
        sig = jax.nn.sigmoid(t)

        pse_blk = -jnp.sum(lqn + sig * t)
        commit_blk = jnp.sum((r - qh) ** 2)
        sums_sc[0, q] += pse_blk
        sums_sc[1, q] += commit_blk

        lqp = lqn + t
        la = jnp.zeros((t.shape[0], 32), jnp.float32)
        for d in range(_HI_BITS):
            la = la + jnp.where(hi_bit[d], lqp[:, d:d + 1], lqn[:, d:d + 1])
        lb = jnp.zeros((t.shape[0], 128), jnp.float32)
        for d in range(_LO_BITS):
            dd = _HI_BITS + d
            lb = lb + jnp.where(lo_bit[d], lqp[:, dd:dd + 1], lqn[:, dd:dd + 1])
        pa = jnp.exp(la)
        pb = jnp.exp(lb)
        avg_sc[q] += jax.lax.dot_general(
            pa, pb, (((0,), (0,)), ((), ())),
            preferred_element_type=jnp.float32,
            precision=jax.lax.Precision.HIGHEST)

        qtot = qtot + qst
        r = r - qst

    idx_ref[...] = jnp.concatenate(idx_cols, axis=1)
    out_ref[...] = jnp.dot(qtot, wout_ref[...],
                           preferred_element_type=jnp.float32) + bout_ref[...]

    @pl.when(i == n_steps - 1)
    def _finalize():
        n = jnp.float32(n_rows_total)
        for q in range(_NUM_Q):
            avg = avg_sc[q] / n
            ce = -jnp.sum(avg * jnp.log(jnp.maximum(avg, _EPS)))
            pse = sums_sc[0, q] / n
            commit = sums_sc[1, q] / (n * _CB_DIM)
            loss_ref[q] = _ENTROPY_W * (pse - ce) + _COMMIT_W * commit


def kernel(x, W_in, b_in, W_out, b_out):
    b, s, d = x.shape
    n = b * s
    n_steps = 8
    rows = n // n_steps
    xf = x.reshape(n, d)

    kern = functools.partial(_lfq_block_kernel,
                             n_rows_total=n, n_steps=n_steps)
    out, idx, losses = pl.pallas_call(
        kern,
        grid=(n_steps,),
        in_specs=[
            pl.BlockSpec((rows, d), lambda i: (i, 0)),
            pl.BlockSpec((d, _CB_DIM), lambda i: (0, 0)),
            pl.BlockSpec((1, _CB_DIM), lambda i: (0, 0)),
            pl.BlockSpec((_CB_DIM, d), lambda i: (0, 0)),
            pl.BlockSpec((1, d), lambda i: (0, 0)),
        ],
        out_specs=[
            pl.BlockSpec((rows, d), lambda i: (i, 0)),
            pl.BlockSpec((rows, _NUM_Q), lambda i: (i, 0)),
            pl.BlockSpec(memory_space=pltpu.SMEM),
        ],
        out_shape=[
            jax.ShapeDtypeStruct((n, d), jnp.float32),
            jax.ShapeDtypeStruct((n, _NUM_Q), jnp.int32),
            jax.ShapeDtypeStruct((_NUM_Q,), jnp.float32),
        ],
        scratch_shapes=[
            pltpu.VMEM((_NUM_Q, 32, 128), jnp.float32),
            pltpu.SMEM((2, _NUM_Q), jnp.float32),
        ],
    )(xf, W_in, b_in.reshape(1, _CB_DIM), W_out, b_out.reshape(1, d))

    return (out.reshape(b, s, d), idx.reshape(b, s, _NUM_Q), losses)


# speedup vs baseline: 4.0828x; 2.4180x over previous
import functools

import numpy as np

import jax
import jax.numpy as jnp
from jax.experimental import pallas as pl
from jax.experimental.pallas import tpu as pltpu

_NUM_Q = 4
_CB_DIM = 12
_HI_BITS = 5
_LO_BITS = 7
_EPS = 1e-5
_INV_TEMP = 100.0
_ENTROPY_W = 0.1
_COMMIT_W = 0.25
_HIGH = jax.lax.Precision.HIGHEST


def _log_prob_mats():
    m_hi = np.zeros((2 * _NUM_Q * _CB_DIM, _NUM_Q * 32), np.float32)
    m_lo = np.zeros((2 * _NUM_Q * _CB_DIM, _NUM_Q * 128), np.float32)
    for q in range(_NUM_Q):
        for d in range(_CB_DIM):
            j = q * _CB_DIM + d
            if d < _HI_BITS:
                for h in range(32):
                    bit = (h >> (_HI_BITS - 1 - d)) & 1
                    m_hi[j + (0 if bit else 48), q * 32 + h] = 1.0
            else:
                for l in range(128):
                    bit = (l >> (11 - d)) & 1
                    m_lo[j + (0 if bit else 48), q * 128 + l] = 1.0
    return m_hi, m_lo


def _idx_mat():
    m = np.zeros((_NUM_Q * _CB_DIM, 128), np.float32)
    for q in range(_NUM_Q):
        for d in range(_CB_DIM):
            m[q * _CB_DIM + d, q] = 2.0 ** (10 - d + q)
    return m


_M_HI, _M_LO = _log_prob_mats()
_M_IDX = _idx_mat()


def _lfq_block_kernel(x_ref, win_ref, bin_ref, w4_ref, bout_ref,
                      mhi_ref, mlo_ref, midx_ref,
                      out_ref, idx_ref, loss_ref,
                      avg_sc, sums_sc, *, n_rows_total, n_steps):
    i = pl.program_id(0)

    @pl.when(i == 0)
    def _init():
        avg_sc[...] = jnp.zeros_like(avg_sc)
        sums_sc[...] = jnp.zeros_like(sums_sc)

    xb = x_ref[...]
    xp = jnp.dot(xb, win_ref[...],
                 preferred_element_type=jnp.float32) + bin_ref[...]

    r = xp
    qh_list, lqp_list, lqn_list, u_list, c2_list = [], [], [], [], []
    for q in range(_NUM_Q):
        scale = 2.0 ** (-q)
        qh = jnp.where(r > 0, scale, -scale)
        w = qh - r
        t = (4.0 * _INV_TEMP * scale) * r
        sp = jnp.maximum(t, 0.0) + jnp.log1p(jnp.exp(-jnp.abs(t)))
        lqn = -sp
        sig = jax.nn.sigmoid(t)
        u_list.append(lqn + sig * t)
        c2_list.append(w * w)
        lqp_list.append(lqn + t)
        lqn_list.append(lqn)
        qh_list.append(qh)
        r = r - (r + w)

    lq = jnp.concatenate(lqp_list + lqn_list, axis=1)
    qhs = jnp.concatenate(qh_list, axis=1)
    stats = jnp.concatenate(u_list + c2_list, axis=1)
    sums_sc[...] += jnp.sum(stats, axis=0, keepdims=True)

    la = jnp.dot(lq, mhi_ref[...], precision=_HIGH,
                 preferred_element_type=jnp.float32)
    lb = jnp.dot(lq, mlo_ref[...], precision=_HIGH,
                 preferred_element_type=jnp.float32)
    pa = jnp.exp(la)
    pb = jnp.exp(lb)
    for q in range(_NUM_Q):
        avg_sc[q] += jax.lax.dot_general(
            pa[:, q * 32:(q + 1) * 32], pb[:, q * 128:(q + 1) * 128],
            (((0,), (0,)), ((), ())), precision=_HIGH,
            preferred_element_type=jnp.float32)

    idxf = jnp.dot(qhs, midx_ref[...],
                   preferred_element_type=jnp.float32)
    idx_ref[...] = (idxf[:, :_NUM_Q] + 2047.5).astype(jnp.int32)
    out_ref[...] = jnp.dot(qhs, w4_ref[...], precision=_HIGH,
                           preferred_element_type=jnp.float32) + bout_ref[...]

    @pl.when(i == n_steps - 1)
    def _finalize():
        n = jnp.float32(n_rows_total)
        sv = sums_sc[...]
        for q in range(_NUM_Q):
            avg = avg_sc[q] / n
            ce = -jnp.sum(avg * jnp.log(jnp.maximum(avg, _EPS)))
            pse = -jnp.sum(sv[:, q * _CB_DIM:(q + 1) * _CB_DIM]) / n
            commit = jnp.sum(
                sv[:, 48 + q * _CB_DIM:48 + (q + 1) * _CB_DIM]) / (n * _CB_DIM)
            loss_ref[q] = _ENTROPY_W * (pse - ce) + _COMMIT_W * commit


def kernel(x, W_in, b_in, W_out, b_out):
    b, s, d = x.shape
    n = b * s
    n_steps = 8
    rows = n // n_steps
    xf = x.reshape(n, d)
    w4 = jnp.concatenate([W_out] * _NUM_Q, axis=0)

    kern = functools.partial(_lfq_block_kernel,
                             n_rows_total=n, n_steps=n_steps)
    out, idx, losses = pl.pallas_call(
        kern,
        grid=(n_steps,),
        in_specs=[
            pl.BlockSpec((rows, d), lambda i: (i, 0)),
            pl.BlockSpec((d, _CB_DIM), lambda i: (0, 0)),
            pl.BlockSpec((1, _CB_DIM), lambda i: (0, 0)),
            pl.BlockSpec((_NUM_Q * _CB_DIM, d), lambda i: (0, 0)),
            pl.BlockSpec((1, d), lambda i: (0, 0)),
            pl.BlockSpec((96, _NUM_Q * 32), lambda i: (0, 0)),
            pl.BlockSpec((96, _NUM_Q * 128), lambda i: (0, 0)),
            pl.BlockSpec((_NUM_Q * _CB_DIM, 128), lambda i: (0, 0)),
        ],
        out_specs=[
            pl.BlockSpec((rows, d), lambda i: (i, 0)),
            pl.BlockSpec((rows, _NUM_Q), lambda i: (i, 0)),
            pl.BlockSpec(memory_space=pltpu.SMEM),
        ],
        out_shape=[
            jax.ShapeDtypeStruct((n, d), jnp.float32),
            jax.ShapeDtypeStruct((n, _NUM_Q), jnp.int32),
            jax.ShapeDtypeStruct((_NUM_Q,), jnp.float32),
        ],
        scratch_shapes=[
            pltpu.VMEM((_NUM_Q, 32, 128), jnp.float32),
            pltpu.VMEM((1, 96), jnp.float32),
        ],
    )(xf, W_in, b_in.reshape(1, _CB_DIM), w4, b_out.reshape(1, d),
      jnp.asarray(_M_HI), jnp.asarray(_M_LO), jnp.asarray(_M_IDX))

    return (out.reshape(b, s, d), idx.reshape(b, s, _NUM_Q), losses)


# speedup vs baseline: 6.1940x; 1.5171x over previous
import functools

import numpy as np

import jax
import jax.numpy as jnp
from jax.experimental import pallas as pl
from jax.experimental.pallas import tpu as pltpu

_NUM_Q = 4
_CB_DIM = 12
_HI_BITS = 5
_LO_BITS = 7
_EPS = 1e-5
_INV_TEMP = 100.0
_ENTROPY_W = 0.1
_COMMIT_W = 0.25


def _log_prob_mats():
    m_hi = np.zeros((2 * _NUM_Q * _CB_DIM, _NUM_Q * 32), np.float32)
    m_lo = np.zeros((2 * _NUM_Q * _CB_DIM, _NUM_Q * 128), np.float32)
    for q in range(_NUM_Q):
        for d in range(_CB_DIM):
            j = q * _CB_DIM + d
            if d < _HI_BITS:
                for h in range(32):
                    bit = (h >> (_HI_BITS - 1 - d)) & 1
                    m_hi[j + (0 if bit else 48), q * 32 + h] = 1.0
            else:
                for l in range(128):
                    bit = (l >> (11 - d)) & 1
                    m_lo[j + (0 if bit else 48), q * 128 + l] = 1.0
    return m_hi, m_lo


def _idx_mat():
    m = np.zeros((_NUM_Q * _CB_DIM, 128), np.float32)
    for q in range(_NUM_Q):
        for d in range(_CB_DIM):
            m[q * _CB_DIM + d, q] = 2.0 ** (10 - d + q)
    return m


_M_HI, _M_LO = _log_prob_mats()
_M_IDX = _idx_mat()
_T_ROW = np.repeat(4.0 * _INV_TEMP * (2.0 ** -np.arange(_NUM_Q)),
                   _CB_DIM).astype(np.float32)[None, :]


def _bf16_split(v):
    hi = v.astype(jnp.bfloat16).astype(jnp.float32)
    return hi, v - hi


def _lfq_block_kernel(x_ref, win_ref, bin_ref, rhs_hi_ref, w4lo_ref, bout_ref,
                      mhi_ref, mlo_ref, trow_ref,
                      out_ref, idx_ref, loss_ref,
                      avg_sc, sums_sc, *, n_rows_total, n_steps):
    i = pl.program_id(0)

    @pl.when(i == 0)
    def _init():
        avg_sc[...] = jnp.zeros_like(avg_sc)
        sums_sc[...] = jnp.zeros_like(sums_sc)

    xb = x_ref[...]
    xp = jnp.dot(xb, win_ref[...],
                 preferred_element_type=jnp.float32) + bin_ref[...]

    r = xp
    r_list, qh_list = [], []
    for q in range(_NUM_Q):
        scale = 2.0 ** (-q)
        qh = jnp.where(r > 0, scale, -scale)
        r_list.append(r)
        qh_list.append(qh)
        r = r - (r + (qh - r))

    r_all = jnp.concatenate(r_list, axis=1)
    qhs = jnp.concatenate(qh_list, axis=1)

    t = r_all * trow_ref[...]
    sp = jnp.maximum(t, 0.0) + jnp.log1p(jnp.exp(-jnp.abs(t)))
    lqn = -sp
    sig = jax.nn.sigmoid(t)
    u = lqn + sig * t
    w = qhs - r_all
    stats = jnp.concatenate([u, w * w], axis=1)
    sums_sc[...] += jnp.sum(stats, axis=0, keepdims=True)

    lq = jnp.concatenate([lqn + t, lqn], axis=1)
    lq_h, lq_l = _bf16_split(lq)
    la = (jnp.dot(lq_h, mhi_ref[...], preferred_element_type=jnp.float32)
          + jnp.dot(lq_l, mhi_ref[...], preferred_element_type=jnp.float32))
    lb = (jnp.dot(lq_h, mlo_ref[...], preferred_element_type=jnp.float32)
          + jnp.dot(lq_l, mlo_ref[...], preferred_element_type=jnp.float32))
    pa = jnp.exp(la)
    pb = jnp.exp(lb)
    pa_h, pa_l = _bf16_split(pa)
    pb_h, pb_l = _bf16_split(pb)
    for q in range(_NUM_Q):
        sa, sb = slice(q * 32, (q + 1) * 32), slice(q * 128, (q + 1) * 128)
        dn = (((0,), (0,)), ((), ()))
        acc = jax.lax.dot_general(pa_h[:, sa], pb_h[:, sb], dn,
                                  preferred_element_type=jnp.float32)
        acc += jax.lax.dot_general(pa_h[:, sa], pb_l[:, sb], dn,
                                   preferred_element_type=jnp.float32)
        acc += jax.lax.dot_general(pa_l[:, sa], pb_h[:, sb], dn,
                                   preferred_element_type=jnp.float32)
        avg_sc[q] += acc

    oh = jnp.dot(qhs, rhs_hi_ref[...],
                 preferred_element_type=jnp.float32)
    ol = jnp.dot(qhs, w4lo_ref[...],
                 preferred_element_type=jnp.float32)
    idx_ref[...] = (oh[:, 768:768 + _NUM_Q] + 2047.5).astype(jnp.int32)
    out_ref[...] = oh[:, :768] + ol + bout_ref[...]

    @pl.when(i == n_steps - 1)
    def _finalize():
        n = jnp.float32(n_rows_total)
        sv = sums_sc[...]
        for q in range(_NUM_Q):
            avg = avg_sc[q] / n
            ce = -jnp.sum(avg * jnp.log(jnp.maximum(avg, _EPS)))
            pse = -jnp.sum(sv[:, q * _CB_DIM:(q + 1) * _CB_DIM]) / n
            commit = jnp.sum(
                sv[:, 48 + q * _CB_DIM:48 + (q + 1) * _CB_DIM]) / (n * _CB_DIM)
            loss_ref[q] = _ENTROPY_W * (pse - ce) + _COMMIT_W * commit


def kernel(x, W_in, b_in, W_out, b_out):
    b, s, d = x.shape
    n = b * s
    n_steps = 8
    rows = n // n_steps
    xf = x.reshape(n, d)
    w4 = jnp.concatenate([W_out] * _NUM_Q, axis=0)
    w4_hi = w4.astype(jnp.bfloat16).astype(jnp.float32)
    w4_lo = w4 - w4_hi
    rhs_hi = jnp.concatenate([w4_hi, jnp.asarray(_M_IDX)], axis=1)

    kern = functools.partial(_lfq_block_kernel,
                             n_rows_total=n, n_steps=n_steps)
    out, idx, losses = pl.pallas_call(
        kern,
        grid=(n_steps,),
        in_specs=[
            pl.BlockSpec((rows, d), lambda i: (i, 0)),
            pl.BlockSpec((d, _CB_DIM), lambda i: (0, 0)),
            pl.BlockSpec((1, _CB_DIM), lambda i: (0, 0)),
            pl.BlockSpec((_NUM_Q * _CB_DIM, d + 128), lambda i: (0, 0)),
            pl.BlockSpec((_NUM_Q * _CB_DIM, d), lambda i: (0, 0)),
            pl.BlockSpec((1, d), lambda i: (0, 0)),
            pl.BlockSpec((96, _NUM_Q * 32), lambda i: (0, 0)),
            pl.BlockSpec((96, _NUM_Q * 128), lambda i: (0, 0)),
            pl.BlockSpec((1, _NUM_Q * _CB_DIM), lambda i: (0, 0)),
        ],
        out_specs=[
            pl.BlockSpec((rows, d), lambda i: (i, 0)),
            pl.BlockSpec((rows, _NUM_Q), lambda i: (i, 0)),
            pl.BlockSpec(memory_space=pltpu.SMEM),
        ],
        out_shape=[
            jax.ShapeDtypeStruct((n, d), jnp.float32),
            jax.ShapeDtypeStruct((n, _NUM_Q), jnp.int32),
            jax.ShapeDtypeStruct((_NUM_Q,), jnp.float32),
        ],
        scratch_shapes=[
            pltpu.VMEM((_NUM_Q, 32, 128), jnp.float32),
            pltpu.VMEM((1, 96), jnp.float32),
        ],
    )(xf, W_in, b_in.reshape(1, _CB_DIM), rhs_hi, w4_lo, b_out.reshape(1, d),
      jnp.asarray(_M_HI), jnp.asarray(_M_LO), jnp.asarray(_T_ROW))

    return (out.reshape(b, s, d), idx.reshape(b, s, _NUM_Q), losses)


# speedup vs baseline: 6.8684x; 1.1089x over previous
import functools

import numpy as np

import jax
import jax.numpy as jnp
from jax.experimental import pallas as pl
from jax.experimental.pallas import tpu as pltpu

_NUM_Q = 4
_CB_DIM = 12
_HI_BITS = 5
_LO_BITS = 7
_EPS = 1e-5
_INV_TEMP = 100.0
_ENTROPY_W = 0.1
_COMMIT_W = 0.25


def _log_prob_mats():
    m_hi = np.zeros((2 * _NUM_Q * _CB_DIM, _NUM_Q * 32), np.float32)
    m_lo = np.zeros((2 * _NUM_Q * _CB_DIM, _NUM_Q * 128), np.float32)
    for q in range(_NUM_Q):
        for d in range(_CB_DIM):
            j = q * _CB_DIM + d
            if d < _HI_BITS:
                for h in range(32):
                    bit = (h >> (_HI_BITS - 1 - d)) & 1
                    m_hi[j + (0 if bit else 48), q * 32 + h] = 1.0
            else:
                for l in range(128):
                    bit = (l >> (11 - d)) & 1
                    m_lo[j + (0 if bit else 48), q * 128 + l] = 1.0
    return m_hi, m_lo


def _idx_mat():
    m = np.zeros((_NUM_Q * _CB_DIM, 128), np.float32)
    for q in range(_NUM_Q):
        for d in range(_CB_DIM):
            m[q * _CB_DIM + d, q] = 2.0 ** (10 - d + q)
    return m


_M_HI, _M_LO = _log_prob_mats()
_M_IDX = _idx_mat()
_T_ROW = np.repeat(4.0 * _INV_TEMP * (2.0 ** -np.arange(_NUM_Q)),
                   _CB_DIM).astype(np.float32)[None, :]


def _bf16_split(v):
    hi = v.astype(jnp.bfloat16).astype(jnp.float32)
    return hi, v - hi


def _lfq_block_kernel(x_ref, win_ref, bin_ref, rhs_ref,
                      mhi_ref, mlo_ref, trow_ref,
                      out_ref, idx_ref, loss_ref,
                      avg_sc, sums_sc, *, n_rows_total, n_steps):
    i = pl.program_id(0)

    @pl.when(i == 0)
    def _init():
        avg_sc[...] = jnp.zeros_like(avg_sc)
        sums_sc[...] = jnp.zeros_like(sums_sc)

    xb = x_ref[...]
    xp = jnp.dot(xb, win_ref[...],
                 preferred_element_type=jnp.float32) + bin_ref[...]

    r = xp
    r_list = []
    for q in range(_NUM_Q):
        scale = 2.0 ** (-q)
        qh = jnp.where(r > 0, scale, -scale)
        r_list.append(r)
        r = r - (r + (qh - r))

    r_all = jnp.concatenate(r_list, axis=1)
    srow = trow_ref[...] * (1.0 / (4.0 * _INV_TEMP))
    qhs = jnp.where(r_all > 0, srow, -srow)

    t = r_all * trow_ref[...]
    sp = jnp.maximum(t, 0.0) + jnp.log1p(jnp.exp(-jnp.abs(t)))
    lqn = -sp
    sig = jax.nn.sigmoid(t)
    u = lqn + sig * t
    w = qhs - r_all
    stats = jnp.concatenate([u, w * w], axis=1)
    sums_sc[...] += jnp.sum(stats, axis=0, keepdims=True)

    lq = jnp.concatenate([lqn + t, lqn], axis=1)
    lq_h, lq_l = _bf16_split(lq)
    lq2 = jnp.concatenate([lq_h, lq_l], axis=1)
    la = jnp.dot(lq2, mhi_ref[...], preferred_element_type=jnp.float32)
    lb = jnp.dot(lq2, mlo_ref[...], preferred_element_type=jnp.float32)
    pa = jnp.exp(la)
    pb = jnp.exp(lb)
    pa_h, pa_l = _bf16_split(pa)
    pb_h, pb_l = _bf16_split(pb)
    for q in range(_NUM_Q):
        sa, sb = slice(q * 32, (q + 1) * 32), slice(q * 128, (q + 1) * 128)
        dn = (((0,), (0,)), ((), ()))
        acc = jax.lax.dot_general(pa_h[:, sa], pb_h[:, sb], dn,
                                  preferred_element_type=jnp.float32)
        acc += jax.lax.dot_general(pa_h[:, sa], pb_l[:, sb], dn,
                                   preferred_element_type=jnp.float32)
        acc += jax.lax.dot_general(pa_l[:, sa], pb_h[:, sb], dn,
                                   preferred_element_type=jnp.float32)
        avg_sc[q] += acc

    lhs = jnp.concatenate(
        [qhs, qhs, jnp.ones((qhs.shape[0], 1), jnp.float32)], axis=1)
    oh = jnp.dot(lhs, rhs_ref[...],
                 preferred_element_type=jnp.float32)
    idx_ref[...] = (oh[:, 768:768 + _NUM_Q] + 2047.5).astype(jnp.int32)
    out_ref[...] = oh[:, :768]

    @pl.when(i == n_steps - 1)
    def _finalize():
        n = jnp.float32(n_rows_total)
        sv = sums_sc[...]
        for q in range(_NUM_Q):
            avg = avg_sc[q] / n
            ce = -jnp.sum(avg * jnp.log(jnp.maximum(avg, _EPS)))
            pse = -jnp.sum(sv[:, q * _CB_DIM:(q + 1) * _CB_DIM]) / n
            commit = jnp.sum(
                sv[:, 48 + q * _CB_DIM:48 + (q + 1) * _CB_DIM]) / (n * _CB_DIM)
            loss_ref[q] = _ENTROPY_W * (pse - ce) + _COMMIT_W * commit


def kernel(x, W_in, b_in, W_out, b_out):
    b, s, d = x.shape
    n = b * s
    n_steps = 8
    rows = n // n_steps
    xf = x.reshape(n, d)
    w4 = jnp.concatenate([W_out] * _NUM_Q, axis=0)
    w4_hi = w4.astype(jnp.bfloat16).astype(jnp.float32)
    w4_lo = w4 - w4_hi
    z48x128 = jnp.zeros((_NUM_Q * _CB_DIM, 128), jnp.float32)
    rhs_big = jnp.concatenate([
        jnp.concatenate([w4_hi, jnp.asarray(_M_IDX)], axis=1),
        jnp.concatenate([w4_lo, z48x128], axis=1),
        jnp.concatenate([b_out.reshape(1, d),
                         jnp.zeros((1, 128), jnp.float32)], axis=1),
    ], axis=0)

    kern = functools.partial(_lfq_block_kernel,
                             n_rows_total=n, n_steps=n_steps)
    out, idx, losses = pl.pallas_call(
        kern,
        grid=(n_steps,),
        in_specs=[
            pl.BlockSpec((rows, d), lambda i: (i, 0)),
            pl.BlockSpec((d, _CB_DIM), lambda i: (0, 0)),
            pl.BlockSpec((1, _CB_DIM), lambda i: (0, 0)),
            pl.BlockSpec((97, d + 128), lambda i: (0, 0)),
            pl.BlockSpec((192, _NUM_Q * 32), lambda i: (0, 0)),
            pl.BlockSpec((192, _NUM_Q * 128), lambda i: (0, 0)),
            pl.BlockSpec((1, _NUM_Q * _CB_DIM), lambda i: (0, 0)),
        ],
        out_specs=[
            pl.BlockSpec((rows, d), lambda i: (i, 0)),
            pl.BlockSpec((rows, _NUM_Q), lambda i: (i, 0)),
            pl.BlockSpec(memory_space=pltpu.SMEM),
        ],
        out_shape=[
            jax.ShapeDtypeStruct((n, d), jnp.float32),
            jax.ShapeDtypeStruct((n, _NUM_Q), jnp.int32),
            jax.ShapeDtypeStruct((_NUM_Q,), jnp.float32),
        ],
        scratch_shapes=[
            pltpu.VMEM((_NUM_Q, 32, 128), jnp.float32),
            pltpu.VMEM((1, 96), jnp.float32),
        ],
    )(xf, W_in, b_in.reshape(1, _CB_DIM), rhs_big,
      jnp.asarray(np.vstack([_M_HI, _M_HI])),
      jnp.asarray(np.vstack([_M_LO, _M_LO])),
      jnp.asarray(_T_ROW))

    return (out.reshape(b, s, d), idx.reshape(b, s, _NUM_Q), losses)


# speedup vs baseline: 6.9554x; 1.0127x over previous
import functools

import numpy as np

import jax
import jax.numpy as jnp
from jax.experimental import pallas as pl
from jax.experimental.pallas import tpu as pltpu

_NUM_Q = 4
_CB_DIM = 12
_HI_BITS = 5
_LO_BITS = 7
_EPS = 1e-5
_INV_TEMP = 100.0
_ENTROPY_W = 0.1
_COMMIT_W = 0.25


def _log_prob_mats():
    m_hi = np.zeros((2 * _NUM_Q * _CB_DIM, _NUM_Q * 32), np.float32)
    m_lo = np.zeros((2 * _NUM_Q * _CB_DIM, _NUM_Q * 128), np.float32)
    for q in range(_NUM_Q):
        for d in range(_CB_DIM):
            j = q * _CB_DIM + d
            if d < _HI_BITS:
                for h in range(32):
                    bit = (h >> (_HI_BITS - 1 - d)) & 1
                    m_hi[j + (0 if bit else 48), q * 32 + h] = 1.0
            else:
                for l in range(128):
                    bit = (l >> (11 - d)) & 1
                    m_lo[j + (0 if bit else 48), q * 128 + l] = 1.0
    return m_hi, m_lo


def _idx_mat():
    m = np.zeros((_NUM_Q * _CB_DIM, 128), np.float32)
    for q in range(_NUM_Q):
        for d in range(_CB_DIM):
            m[q * _CB_DIM + d, q] = 2.0 ** (10 - d + q)
    return m


_M_HI, _M_LO = _log_prob_mats()
_M_IDX = _idx_mat()
_T_ROW = np.repeat(4.0 * _INV_TEMP * (2.0 ** -np.arange(_NUM_Q)),
                   _CB_DIM).astype(np.float32)[None, :]


def _bf16_split(v):
    hi = v.astype(jnp.bfloat16).astype(jnp.float32)
    return hi, v - hi


def _lfq_block_kernel(x_ref, win_ref, bin_ref, rhs_ref,
                      mhi_ref, mlo_ref, trow_ref,
                      out_ref, idx_ref, loss_ref,
                      avg_sc, sums_sc, *, n_rows_total, n_steps):
    i = pl.program_id(0)

    @pl.when(i == 0)
    def _init():
        avg_sc[...] = jnp.zeros_like(avg_sc)
        sums_sc[...] = jnp.zeros_like(sums_sc)

    xb = x_ref[...]
    xp = jnp.dot(xb, win_ref[...],
                 preferred_element_type=jnp.float32) + bin_ref[...]

    r = xp
    r_list = []
    for q in range(_NUM_Q):
        scale = 2.0 ** (-q)
        qh = jnp.where(r > 0, scale, -scale)
        r_list.append(r)
        r = r - (r + (qh - r))

    r_all = jnp.concatenate(r_list, axis=1)
    srow = trow_ref[...] * (1.0 / (4.0 * _INV_TEMP))
    qhs = jnp.where(r_all > 0, srow, -srow)

    t = r_all * trow_ref[...]
    sp = jnp.maximum(t, 0.0) + jnp.log1p(jnp.exp(-jnp.abs(t)))
    lqn = -sp
    sig = jax.nn.sigmoid(t)
    u = lqn + sig * t
    w = qhs - r_all
    stats = jnp.concatenate([u, w * w], axis=1)
    sums_sc[...] += jnp.sum(stats, axis=0, keepdims=True)

    lq = jnp.concatenate([lqn + t, lqn], axis=1)
    lq_h, lq_l = _bf16_split(lq)
    lq2 = jnp.concatenate([lq_h, lq_l], axis=1)
    la = jnp.dot(lq2, mhi_ref[...], preferred_element_type=jnp.float32)
    lb = jnp.dot(lq2, mlo_ref[...], preferred_element_type=jnp.float32)
    pa = jnp.exp(la)
    pb = jnp.exp(lb)
    pa_h, pa_l = _bf16_split(pa)
    pb_h, pb_l = _bf16_split(pb)
    for q in range(_NUM_Q):
        sa, sb = slice(q * 32, (q + 1) * 32), slice(q * 128, (q + 1) * 128)
        dn = (((0,), (0,)), ((), ()))
        acc = jax.lax.dot_general(pa_h[:, sa], pb_h[:, sb], dn,
                                  preferred_element_type=jnp.float32)
        acc += jax.lax.dot_general(pa_h[:, sa], pb_l[:, sb], dn,
                                   preferred_element_type=jnp.float32)
        acc += jax.lax.dot_general(pa_l[:, sa], pb_h[:, sb], dn,
                                   preferred_element_type=jnp.float32)
        avg_sc[q] += acc

    ones = jnp.ones((qhs.shape[0], 1), jnp.float32)
    lhs = jnp.concatenate([qhs, qhs, ones, ones], axis=1)
    oh = jnp.dot(lhs, rhs_ref[...],
                 preferred_element_type=jnp.float32)
    idx_ref[...] = (oh[:, 768:768 + _NUM_Q] + 2047.5).astype(jnp.int32)
    out_ref[...] = oh[:, :768]

    @pl.when(i == n_steps - 1)
    def _finalize():
        n = jnp.float32(n_rows_total)
        sv = sums_sc[...]
        for q in range(_NUM_Q):
            avg = avg_sc[q] / n
            ce = -jnp.sum(avg * jnp.log(jnp.maximum(avg, _EPS)))
            pse = -jnp.sum(sv[:, q * _CB_DIM:(q + 1) * _CB_DIM]) / n
            commit = jnp.sum(
                sv[:, 48 + q * _CB_DIM:48 + (q + 1) * _CB_DIM]) / (n * _CB_DIM)
            loss_ref[q] = _ENTROPY_W * (pse - ce) + _COMMIT_W * commit


def kernel(x, W_in, b_in, W_out, b_out):
    b, s, d = x.shape
    n = b * s
    n_steps = 8
    rows = n // n_steps
    xf = x.reshape(n, d)
    w4 = jnp.concatenate([W_out] * _NUM_Q, axis=0)
    w4_hi = w4.astype(jnp.bfloat16).astype(jnp.float32)
    w4_lo = w4 - w4_hi
    z48x128 = jnp.zeros((_NUM_Q * _CB_DIM, 128), jnp.float32)
    bo = b_out.reshape(1, d)
    bo_hi = bo.astype(jnp.bfloat16).astype(jnp.float32)
    z1x128 = jnp.zeros((1, 128), jnp.float32)
    rhs_big = jnp.concatenate([
        jnp.concatenate([w4_hi, jnp.asarray(_M_IDX)], axis=1),
        jnp.concatenate([w4_lo, z48x128], axis=1),
        jnp.concatenate([bo_hi, z1x128], axis=1),
        jnp.concatenate([bo - bo_hi, z1x128], axis=1),
    ], axis=0)

    kern = functools.partial(_lfq_block_kernel,
                             n_rows_total=n, n_steps=n_steps)
    out, idx, losses = pl.pallas_call(
        kern,
        grid=(n_steps,),
        in_specs=[
            pl.BlockSpec((rows, d), lambda i: (i, 0)),
            pl.BlockSpec((d, _CB_DIM), lambda i: (0, 0)),
            pl.BlockSpec((1, _CB_DIM), lambda i: (0, 0)),
            pl.BlockSpec((98, d + 128), lambda i: (0, 0)),
            pl.BlockSpec((192, _NUM_Q * 32), lambda i: (0, 0)),
            pl.BlockSpec((192, _NUM_Q * 128), lambda i: (0, 0)),
            pl.BlockSpec((1, _NUM_Q * _CB_DIM), lambda i: (0, 0)),
        ],
        out_specs=[
            pl.BlockSpec((rows, d), lambda i: (i, 0)),
            pl.BlockSpec((rows, _NUM_Q), lambda i: (i, 0)),
            pl.BlockSpec(memory_space=pltpu.SMEM),
        ],
        out_shape=[
            jax.ShapeDtypeStruct((n, d), jnp.float32),
            jax.ShapeDtypeStruct((n, _NUM_Q), jnp.int32),
            jax.ShapeDtypeStruct((_NUM_Q,), jnp.float32),
        ],
        scratch_shapes=[
            pltpu.VMEM((_NUM_Q, 32, 128), jnp.float32),
            pltpu.VMEM((1, 96), jnp.float32),
        ],
    )(xf, W_in, b_in.reshape(1, _CB_DIM), rhs_big,
      jnp.asarray(np.vstack([_M_HI, _M_HI])),
      jnp.asarray(np.vstack([_M_LO, _M_LO])),
      jnp.asarray(_T_ROW))

    return (out.reshape(b, s, d), idx.reshape(b, s, _NUM_Q), losses)
